# Initial kernel scaffold; baseline (speedup 1.0000x reference)
import jax
import jax.numpy as jnp
from jax.experimental import pallas as pl


def kernel(x, token_table, pos_table):
    raise NotImplementedError("write your pallas kernel here")


# speedup vs baseline: 1.4249x; 1.4249x over previous
import functools

import jax
import jax.numpy as jnp
from jax import lax
from jax.experimental import pallas as pl
from jax.experimental.pallas import tpu as pltpu
from jax.experimental.pallas import tpu_sc as plsc

_L = 200
_D = 32
_B = 4096
_ROWS = _B * _L
_NW = 32
_ROWS_PER_W = _ROWS // _NW
_CHUNK = 1600
_SEQS_PER_CHUNK = _CHUNK // _L
_CHUNKS_PER_W = _ROWS_PER_W // _CHUNK
_GW = 100
_NG = _CHUNK // _GW


def _sc_embed(xi, token_table, pos_table):
    mesh = plsc.VectorSubcoreMesh(core_axis_name="c", subcore_axis_name="s")

    @functools.partial(
        pl.kernel,
        out_type=jax.ShapeDtypeStruct((_ROWS, _D), jnp.float32),
        mesh=mesh,
        compiler_params=pltpu.CompilerParams(use_tc_tiling_on_sc=False),
        scratch_types=[
            pltpu.VMEM((_NG, _GW), jnp.int32),
            pltpu.VMEM((_CHUNK, _D), jnp.float32),
            pltpu.VMEM((_L, _D), jnp.float32),
            pltpu.SemaphoreType.DMA,
        ],
    )
    def k(x_hbm, tok_hbm, pos_hbm, out_hbm, idx_v, rows_v, pos_v, sem):
        wid = lax.axis_index("s") * 2 + lax.axis_index("c")
        pltpu.sync_copy(pos_hbm, pos_v)
        w_row0 = wid * (_ROWS_PER_W // _GW)

        @pl.loop(0, _CHUNKS_PER_W)
        def _chunk(ci):
            r0 = w_row0 + ci * _NG
            pltpu.sync_copy(x_hbm.at[pl.ds(r0, _NG)], idx_v)
            copies = [
                pltpu.async_copy(
                    tok_hbm.at[idx_v.at[j]],
                    rows_v.at[pl.ds(j * _GW, _GW)],
                    sem,
                )
                for j in range(_NG)
            ]
            for cp in copies:
                cp.wait()

            @pl.loop(0, _L)
            def _row(r):
                for h in range(2):
                    pv = pos_v[pl.ds(r, 1), pl.ds(16 * h, 16)]
                    for s in range(_SEQS_PER_CHUNK):
                        slc = (pl.ds(s * _L + r, 1), pl.ds(16 * h, 16))
                        rows_v[slc] = rows_v[slc] + pv

            pltpu.sync_copy(rows_v, out_hbm.at[pl.ds(r0 * _GW, _CHUNK)])

    return k(xi, token_table, pos_table)


def kernel(x, token_table, pos_table):
    b, l = x.shape
    xi = x.reshape(_ROWS // _GW, _GW).astype(jnp.int32)
    out = _sc_embed(xi, token_table, pos_table)
    return out.reshape(b, l, _D)
